# baseline (speedup 1.0000x reference)
import jax
import jax.numpy as jnp
from jax import lax
from jax.experimental import pallas as pl
from jax.experimental.pallas import tpu as pltpu, tpu_sc as plsc

_L = 16
_NBKT = 256


def _make_sc_kernel(b, n, k, nc=2, ns=16, interpret=False):
    nw = nc * ns
    rpw = b // nw
    nv = n // _L
    mesh = plsc.VectorSubcoreMesh(
        core_axis_name="c", subcore_axis_name="s",
        num_cores=nc, num_subcores=ns)

    def body(x_hbm, out_hbm, row_v, cand_v, out_v, hist_v, tot_v, kept_v,
             sem_out, sem_in0, sem_in1, sem_in2, sem_in3):
        cid = lax.axis_index("c")
        sid = lax.axis_index("s")
        wid = sid * nc + cid
        base = wid * rpw
        lanes = lax.broadcasted_iota(jnp.int32, (_L,), 0)
        laneoff = lanes * _NBKT
        laneoff128 = lanes * _NBKT + 128
        zeros16 = jnp.zeros((_L,), jnp.int32)
        ones16 = jnp.ones((_L,), jnp.int32)
        fzeros16 = jnp.zeros((_L,), jnp.float32)
        kk = jnp.int32(k)

        def monotone_key(v):
            xi = plsc.bitcast(v, jnp.int32)
            return xi ^ ((xi >> 31) & jnp.int32(0x7FFFFFFF))

        sems_in = (sem_in0, sem_in1, sem_in2, sem_in3)
        nch = 4
        chn = n // nch
        nvch = nv // nch

        def issue_in(i):
            return [
                pltpu.async_copy(
                    x_hbm.at[base + i, pl.ds(c * chn, chn)],
                    row_v.at[pl.ds(c * chn, chn)], sems_in[c])
                for c in range(nch)
            ]

        descs_in = issue_in(0)

        @plsc.parallel_loop(0, nv, unroll=8)
        def _(j):
            out_v[pl.ds(j * _L, _L)] = fzeros16

        @plsc.parallel_loop(0, _NBKT, unroll=8)
        def _(j):
            hist_v[pl.ds(j * _L, _L)] = zeros16

        desc_out = None
        for i in range(rpw):

            for c in range(nch):
                descs_in[c].wait()

                @plsc.parallel_loop(c * nvch, (c + 1) * nvch, unroll=16)
                def _(j):
                    xi = plsc.bitcast(row_v[pl.ds(j * _L, _L)], jnp.int32)
                    bkt = (xi >> 24) ^ ((xi >> 31) & jnp.int32(0x7F))
                    plsc.addupdate_scatter(hist_v, [laneoff128 + bkt],
                                           ones16)

            def t_body(g, _):
                acc = zeros16
                for l in range(_L):
                    sl = pl.ds(l * _NBKT + g * _L, _L)
                    acc = acc + hist_v[sl]
                    hist_v[sl] = zeros16
                tot_v[pl.ds(g * _L, _L)] = acc
                return 0
            lax.fori_loop(0, _NBKT // _L, t_body, 0)

            def nb_body(t, carry):
                hi, cntb = carry
                g = jnp.int32(_NBKT // _L - 1) - t
                tg = tot_v[pl.ds(g * _L, _L)]
                suf = lax.rev(plsc.cumsum(lax.rev(tg, (0,))), (0,)) + hi
                cntb = cntb + jnp.sum((suf >= kk).astype(jnp.int32))
                return (hi + jnp.sum(tg), cntb)
            _, nbcnt = lax.fori_loop(0, _NBKT // _L, nb_body,
                                     (jnp.int32(0), jnp.int32(0)))
            b1 = nbcnt - 1

            def c_body(j, ptr):
                key = monotone_key(row_v[pl.ds(j * _L, _L)])
                bkt = (key >> 24) + 128
                m = bkt >= b1
                cs = plsc.cumsum(m.astype(jnp.int32))
                addr = ptr + cs - 1
                plsc.store_scatter(cand_v, [addr], j * _L + lanes, mask=m)
                plsc.store_scatter(row_v, [addr],
                                   plsc.bitcast(key, jnp.float32), mask=m)
                return ptr + plsc.all_reduce_population_count(m)
            ptr = plsc.parallel_loop(0, nv, carry=zeros16, unroll=8)(c_body)
            m1 = jnp.max(ptr)
            nv_c = (m1 + _L - 1) // _L

            key_top = (b1 - 128) << 24

            def cand_keys(j):
                valid = (j * _L + lanes) < m1
                keyv = plsc.bitcast(row_v[pl.ds(j * _L, _L)], jnp.int32)
                return keyv, valid

            key_top_v = jnp.full((_L,), 1, jnp.int32) * key_top

            def bit_body(t, prefix):
                cand_t = prefix | (jnp.int32(1) << (jnp.int32(23) - t))

                def cnt_body(j, acc):
                    keyv, valid = cand_keys(j)
                    ge = (keyv >= cand_t) & valid
                    return acc + plsc.all_reduce_population_count(ge)
                cnt = plsc.parallel_loop(0, nv_c, carry=zeros16,
                                         unroll=4)(cnt_body)
                return jnp.where(cnt >= kk, cand_t, prefix)
            thr = lax.fori_loop(0, 24, bit_body, key_top_v)

            def gt_body(j, acc):
                keyv, valid = cand_keys(j)
                gt = (keyv > thr) & valid
                return acc + plsc.all_reduce_population_count(gt)
            cgt = plsc.parallel_loop(0, nv_c, carry=zeros16,
                                     unroll=4)(gt_body)
            r = kk - cgt

            def tie_body(j, carry):
                cnt, istar_l = carry
                keyv, valid = cand_keys(j)
                idxv = cand_v[pl.ds(j * _L, _L)]
                eq = (keyv == thr) & valid
                cs = plsc.cumsum(eq.astype(jnp.int32))
                hit = eq & ((cs + cnt) == r)
                istar_l = istar_l + jnp.where(hit, idxv, 0)
                return (cnt + plsc.all_reduce_population_count(eq), istar_l)
            _, istar_l = plsc.parallel_loop(
                0, nv_c, carry=(zeros16, zeros16), unroll=2)(tie_body)
            istar = jnp.sum(istar_l)

            if i > 0:
                desc_out.wait()

                @plsc.parallel_loop(0, k // _L, unroll=4)
                def _(j):
                    idxv = kept_v[pl.ds(j * _L, _L)]
                    plsc.store_scatter(out_v, [idxv], fzeros16)

            def vs_body(j, wptr):
                keyv, valid = cand_keys(j)
                idxv = cand_v[pl.ds(j * _L, _L)]
                xv = plsc.bitcast(monotone_key(
                    plsc.bitcast(keyv, jnp.float32)), jnp.float32)
                keep = valid & ((keyv > thr)
                                | ((keyv == thr) & (idxv <= istar)))
                plsc.store_scatter(out_v, [idxv], xv, mask=keep)
                cs = plsc.cumsum(keep.astype(jnp.int32))
                plsc.store_scatter(kept_v, [wptr + cs - 1], idxv, mask=keep)
                return wptr + plsc.all_reduce_population_count(keep)
            plsc.parallel_loop(0, nv_c, carry=zeros16, unroll=4)(vs_body)

            if i + 1 < rpw:
                descs_in = issue_in(i + 1)
            desc_out = pltpu.async_copy(out_v, out_hbm.at[base + i], sem_out)
        desc_out.wait()

    sck = pl.kernel(
        body,
        out_type=jax.ShapeDtypeStruct((b, n), jnp.float32),
        mesh=mesh,
        scratch_types=[
            pltpu.VMEM((n,), jnp.float32),
            pltpu.VMEM((n,), jnp.int32),
            pltpu.VMEM((n,), jnp.float32),
            pltpu.VMEM((_NBKT * _L,), jnp.int32),
            pltpu.VMEM((_NBKT,), jnp.int32),
            pltpu.VMEM((k,), jnp.int32),
            pltpu.SemaphoreType.DMA,
            pltpu.SemaphoreType.DMA,
            pltpu.SemaphoreType.DMA,
            pltpu.SemaphoreType.DMA,
            pltpu.SemaphoreType.DMA,
        ],
        compiler_params=pltpu.CompilerParams(needs_layout_passes=False),
        interpret=interpret,
    )

    return sck


_kern = _make_sc_kernel(128, 32768, 512)


def kernel(x):
    return _kern(x)


# speedup vs baseline: 1.0106x; 1.0106x over previous
import jax
import jax.numpy as jnp
from jax import lax
from jax.experimental import pallas as pl
from jax.experimental.pallas import tpu as pltpu, tpu_sc as plsc

_L = 16
_NBKT = 256


def _make_sc_kernel(b, n, k, nc=2, ns=16, interpret=False):
    nw = nc * ns
    rpw = b // nw
    nv = n // _L
    mesh = plsc.VectorSubcoreMesh(
        core_axis_name="c", subcore_axis_name="s",
        num_cores=nc, num_subcores=ns)

    def body(x_hbm, out_hbm, row_v, cand_v, out_v, hist_v, tot_v, kept_v,
             sem_out, sem_in0, sem_in1, sem_in2, sem_in3):
        cid = lax.axis_index("c")
        sid = lax.axis_index("s")
        wid = sid * nc + cid
        base = wid * rpw
        lanes = lax.broadcasted_iota(jnp.int32, (_L,), 0)
        laneoff = lanes * _NBKT
        laneoff128 = lanes * _NBKT + 128
        zeros16 = jnp.zeros((_L,), jnp.int32)
        ones16 = jnp.ones((_L,), jnp.int32)
        fzeros16 = jnp.zeros((_L,), jnp.float32)
        kk = jnp.int32(k)

        def monotone_key(v):
            xi = plsc.bitcast(v, jnp.int32)
            return xi ^ ((xi >> 31) & jnp.int32(0x7FFFFFFF))

        sems_in = (sem_in0, sem_in1, sem_in2, sem_in3)
        nch = 4
        chn = n // nch
        nvch = nv // nch

        def issue_in(i):
            return [
                pltpu.async_copy(
                    x_hbm.at[base + i, pl.ds(c * chn, chn)],
                    row_v.at[pl.ds(c * chn, chn)], sems_in[c])
                for c in range(nch)
            ]

        descs_in = issue_in(0)

        @plsc.parallel_loop(0, nv, unroll=8)
        def _(j):
            out_v[pl.ds(j * _L, _L)] = fzeros16

        @plsc.parallel_loop(0, _NBKT, unroll=8)
        def _(j):
            hist_v[pl.ds(j * _L, _L)] = zeros16

        desc_out = None
        for i in range(rpw):
            if i > 0:
                @plsc.parallel_loop(0, _NBKT, unroll=8)
                def _(j):
                    hist_v[pl.ds(j * _L, _L)] = zeros16

            for c in range(nch):
                descs_in[c].wait()

                @plsc.parallel_loop(c * nvch, (c + 1) * nvch, unroll=8)
                def _(j):
                    xi = plsc.bitcast(row_v[pl.ds(j * _L, _L)], jnp.int32)
                    bkt = (xi >> 24) ^ ((xi >> 31) & jnp.int32(0x7F))
                    plsc.addupdate_scatter(hist_v, [laneoff128 + bkt],
                                           ones16)

            def t_body(g, _):
                acc = zeros16
                for l in range(_L):
                    acc = acc + hist_v[pl.ds(l * _NBKT + g * _L, _L)]
                tot_v[pl.ds(g * _L, _L)] = acc
                return 0
            lax.fori_loop(0, _NBKT // _L, t_body, 0)

            def nb_body(t, carry):
                hi, cntb = carry
                g = jnp.int32(_NBKT // _L - 1) - t
                tg = tot_v[pl.ds(g * _L, _L)]
                suf = lax.rev(plsc.cumsum(lax.rev(tg, (0,))), (0,)) + hi
                cntb = cntb + jnp.sum((suf >= kk).astype(jnp.int32))
                return (hi + jnp.sum(tg), cntb)
            _, nbcnt = lax.fori_loop(0, _NBKT // _L, nb_body,
                                     (jnp.int32(0), jnp.int32(0)))
            b1 = nbcnt - 1

            def c_body(j, ptr):
                key = monotone_key(row_v[pl.ds(j * _L, _L)])
                bkt = (key >> 24) + 128
                m = bkt >= b1
                cs = plsc.cumsum(m.astype(jnp.int32))
                addr = ptr + cs - 1
                plsc.store_scatter(cand_v, [addr], j * _L + lanes, mask=m)
                plsc.store_scatter(row_v, [addr],
                                   plsc.bitcast(key, jnp.float32), mask=m)
                return ptr + plsc.all_reduce_population_count(m)
            ptr = plsc.parallel_loop(0, nv, carry=zeros16, unroll=8)(c_body)
            m1 = jnp.max(ptr)
            nv_c = (m1 + _L - 1) // _L

            key_top = (b1 - 128) << 24

            def cand_keys(j):
                valid = (j * _L + lanes) < m1
                keyv = plsc.bitcast(row_v[pl.ds(j * _L, _L)], jnp.int32)
                return keyv, valid

            key_top_v = jnp.full((_L,), 1, jnp.int32) * key_top

            def bit_body(t, prefix):
                cand_t = prefix | (jnp.int32(1) << (jnp.int32(23) - t))

                def cnt_body(j, acc):
                    keyv, valid = cand_keys(j)
                    ge = (keyv >= cand_t) & valid
                    return acc + plsc.all_reduce_population_count(ge)
                cnt = plsc.parallel_loop(0, nv_c, carry=zeros16,
                                         unroll=4)(cnt_body)
                return jnp.where(cnt >= kk, cand_t, prefix)
            thr = lax.fori_loop(0, 24, bit_body, key_top_v)

            def gt_body(j, acc):
                keyv, valid = cand_keys(j)
                gt = (keyv > thr) & valid
                return acc + plsc.all_reduce_population_count(gt)
            cgt = plsc.parallel_loop(0, nv_c, carry=zeros16,
                                     unroll=4)(gt_body)
            r = kk - cgt

            def tie_body(j, carry):
                cnt, istar_l = carry
                keyv, valid = cand_keys(j)
                idxv = cand_v[pl.ds(j * _L, _L)]
                eq = (keyv == thr) & valid
                cs = plsc.cumsum(eq.astype(jnp.int32))
                hit = eq & ((cs + cnt) == r)
                istar_l = istar_l + jnp.where(hit, idxv, 0)
                return (cnt + plsc.all_reduce_population_count(eq), istar_l)
            _, istar_l = plsc.parallel_loop(
                0, nv_c, carry=(zeros16, zeros16), unroll=2)(tie_body)
            istar = jnp.sum(istar_l)

            if i > 0:
                desc_out.wait()

                @plsc.parallel_loop(0, k // _L, unroll=4)
                def _(j):
                    idxv = kept_v[pl.ds(j * _L, _L)]
                    plsc.store_scatter(out_v, [idxv], fzeros16)

            def vs_body(j, wptr):
                keyv, valid = cand_keys(j)
                idxv = cand_v[pl.ds(j * _L, _L)]
                xv = plsc.bitcast(monotone_key(
                    plsc.bitcast(keyv, jnp.float32)), jnp.float32)
                keep = valid & ((keyv > thr)
                                | ((keyv == thr) & (idxv <= istar)))
                plsc.store_scatter(out_v, [idxv], xv, mask=keep)
                cs = plsc.cumsum(keep.astype(jnp.int32))
                plsc.store_scatter(kept_v, [wptr + cs - 1], idxv, mask=keep)
                return wptr + plsc.all_reduce_population_count(keep)
            plsc.parallel_loop(0, nv_c, carry=zeros16, unroll=4)(vs_body)

            if i + 1 < rpw:
                descs_in = issue_in(i + 1)
            desc_out = pltpu.async_copy(out_v, out_hbm.at[base + i], sem_out)
        desc_out.wait()

    sck = pl.kernel(
        body,
        out_type=jax.ShapeDtypeStruct((b, n), jnp.float32),
        mesh=mesh,
        scratch_types=[
            pltpu.VMEM((n,), jnp.float32),
            pltpu.VMEM((n,), jnp.int32),
            pltpu.VMEM((n,), jnp.float32),
            pltpu.VMEM((_NBKT * _L,), jnp.int32),
            pltpu.VMEM((_NBKT,), jnp.int32),
            pltpu.VMEM((k,), jnp.int32),
            pltpu.SemaphoreType.DMA,
            pltpu.SemaphoreType.DMA,
            pltpu.SemaphoreType.DMA,
            pltpu.SemaphoreType.DMA,
            pltpu.SemaphoreType.DMA,
        ],
        compiler_params=pltpu.CompilerParams(needs_layout_passes=False),
        interpret=interpret,
    )

    return sck


_kern = _make_sc_kernel(128, 32768, 512)


def kernel(x):
    return _kern(x)


# speedup vs baseline: 1.0177x; 1.0071x over previous
import jax
import jax.numpy as jnp
from jax import lax
from jax.experimental import pallas as pl
from jax.experimental.pallas import tpu as pltpu, tpu_sc as plsc

_L = 16
_NBKT = 256


def _make_sc_kernel(b, n, k, nc=2, ns=16, interpret=False):
    nw = nc * ns
    rpw = b // nw
    nv = n // _L
    mesh = plsc.VectorSubcoreMesh(
        core_axis_name="c", subcore_axis_name="s",
        num_cores=nc, num_subcores=ns)

    def body(x_hbm, out_hbm, row_v, cand_v, out_v, hist_v, tot_v, kept_v,
             sem_out, sem_in0, sem_in1, sem_in2, sem_in3):
        cid = lax.axis_index("c")
        sid = lax.axis_index("s")
        wid = sid * nc + cid
        base = wid * rpw
        lanes = lax.broadcasted_iota(jnp.int32, (_L,), 0)
        laneoff = lanes * _NBKT
        laneoff128 = lanes * _NBKT + 128
        zeros16 = jnp.zeros((_L,), jnp.int32)
        ones16 = jnp.ones((_L,), jnp.int32)
        fzeros16 = jnp.zeros((_L,), jnp.float32)
        kk = jnp.int32(k)

        def monotone_key(v):
            xi = plsc.bitcast(v, jnp.int32)
            return xi ^ ((xi >> 31) & jnp.int32(0x7FFFFFFF))

        sems_in = (sem_in0, sem_in1, sem_in2, sem_in3)
        nch = 4
        chn = n // nch
        nvch = nv // nch

        def issue_in(i):
            return [
                pltpu.async_copy(
                    x_hbm.at[base + i, pl.ds(c * chn, chn)],
                    row_v.at[pl.ds(c * chn, chn)], sems_in[c])
                for c in range(nch)
            ]

        descs_in = issue_in(0)

        @plsc.parallel_loop(0, nv, unroll=8)
        def _(j):
            out_v[pl.ds(j * _L, _L)] = fzeros16

        @plsc.parallel_loop(0, _NBKT, unroll=8)
        def _(j):
            hist_v[pl.ds(j * _L, _L)] = zeros16

        desc_out = None
        for i in range(rpw):
            if i > 0:
                @plsc.parallel_loop(0, _NBKT, unroll=8)
                def _(j):
                    hist_v[pl.ds(j * _L, _L)] = zeros16

            for c in range(nch):
                descs_in[c].wait()

                @plsc.parallel_loop(c * nvch, (c + 1) * nvch, unroll=8)
                def _(j):
                    xi = plsc.bitcast(row_v[pl.ds(j * _L, _L)], jnp.int32)
                    bkt = (xi >> 24) ^ ((xi >> 31) & jnp.int32(0x7F))
                    plsc.addupdate_scatter(hist_v, [laneoff128 + bkt],
                                           ones16)

            def t_body(g, _):
                acc = zeros16
                for l in range(_L):
                    acc = acc + hist_v[pl.ds(l * _NBKT + g * _L, _L)]
                tot_v[pl.ds(g * _L, _L)] = acc
                return 0
            lax.fori_loop(0, _NBKT // _L, t_body, 0)

            def nb_body(t, carry):
                hi, cntb = carry
                g = jnp.int32(_NBKT // _L - 1) - t
                tg = tot_v[pl.ds(g * _L, _L)]
                suf = lax.rev(plsc.cumsum(lax.rev(tg, (0,))), (0,)) + hi
                cntb = cntb + jnp.sum((suf >= kk).astype(jnp.int32))
                return (hi + jnp.sum(tg), cntb)
            _, nbcnt = lax.fori_loop(0, _NBKT // _L, nb_body,
                                     (jnp.int32(0), jnp.int32(0)))
            b1 = nbcnt - 1

            def c_body(j, ptr):
                key = monotone_key(row_v[pl.ds(j * _L, _L)])
                bkt = (key >> 24) + 128
                m = bkt >= b1
                cs = plsc.cumsum(m.astype(jnp.int32))
                addr = ptr + cs - 1
                plsc.store_scatter(cand_v, [addr], j * _L + lanes, mask=m)
                plsc.store_scatter(row_v, [addr],
                                   plsc.bitcast(key, jnp.float32), mask=m)
                return ptr + plsc.all_reduce_population_count(m)
            ptr = plsc.parallel_loop(0, nv, carry=zeros16, unroll=8)(c_body)
            m1 = jnp.max(ptr)
            nv_c = (m1 + _L - 1) // _L

            key_top = (b1 - 128) << 24

            def cand_keys(j):
                valid = (j * _L + lanes) < m1
                keyv = plsc.bitcast(row_v[pl.ds(j * _L, _L)], jnp.int32)
                return keyv, valid

            key_top_v = jnp.full((_L,), 1, jnp.int32) * key_top

            def bit_body(t, carry):
                prefix, cge = carry
                cand_t = prefix | (jnp.int32(1) << (jnp.int32(23) - t))

                def cnt_body(j, acc):
                    keyv, valid = cand_keys(j)
                    ge = (keyv >= cand_t) & valid
                    return acc + plsc.all_reduce_population_count(ge)
                cnt = plsc.parallel_loop(0, nv_c, carry=zeros16,
                                         unroll=4)(cnt_body)
                take = cnt >= kk
                return (jnp.where(take, cand_t, prefix),
                        jnp.where(take, cnt, cge))
            thr, cge = lax.fori_loop(0, 24, bit_body, (key_top_v, ptr))

            d1 = cge - kk + 1

            def tie_body(j, carry):
                cnt, istar_l = carry
                jj = nv_c - 1 - j
                valid = (jj * _L + lanes) < m1
                keyv = plsc.bitcast(row_v[pl.ds(jj * _L, _L)], jnp.int32)
                idxv = cand_v[pl.ds(jj * _L, _L)]
                eq = (keyv == thr) & valid
                pc = plsc.all_reduce_population_count(eq)
                cs = plsc.cumsum(eq.astype(jnp.int32))
                rrank = pc - cs + eq.astype(jnp.int32) + cnt
                hit = eq & (rrank == d1)
                istar_l = istar_l + jnp.where(hit, idxv, 0)
                return (cnt + pc, istar_l)
            _, istar_l = plsc.parallel_loop(
                0, nv_c, carry=(zeros16, zeros16), unroll=2)(tie_body)
            istar = jnp.sum(istar_l)

            if i > 0:
                desc_out.wait()

                @plsc.parallel_loop(0, k // _L, unroll=4)
                def _(j):
                    idxv = kept_v[pl.ds(j * _L, _L)]
                    plsc.store_scatter(out_v, [idxv], fzeros16)

            def vs_body(j, wptr):
                keyv, valid = cand_keys(j)
                idxv = cand_v[pl.ds(j * _L, _L)]
                xv = plsc.bitcast(monotone_key(
                    plsc.bitcast(keyv, jnp.float32)), jnp.float32)
                keep = valid & ((keyv > thr)
                                | ((keyv == thr) & (idxv <= istar)))
                plsc.store_scatter(out_v, [idxv], xv, mask=keep)
                cs = plsc.cumsum(keep.astype(jnp.int32))
                plsc.store_scatter(kept_v, [wptr + cs - 1], idxv, mask=keep)
                return wptr + plsc.all_reduce_population_count(keep)
            plsc.parallel_loop(0, nv_c, carry=zeros16, unroll=4)(vs_body)

            if i + 1 < rpw:
                descs_in = issue_in(i + 1)
            desc_out = pltpu.async_copy(out_v, out_hbm.at[base + i], sem_out)
        desc_out.wait()

    sck = pl.kernel(
        body,
        out_type=jax.ShapeDtypeStruct((b, n), jnp.float32),
        mesh=mesh,
        scratch_types=[
            pltpu.VMEM((n,), jnp.float32),
            pltpu.VMEM((n,), jnp.int32),
            pltpu.VMEM((n,), jnp.float32),
            pltpu.VMEM((_NBKT * _L,), jnp.int32),
            pltpu.VMEM((_NBKT,), jnp.int32),
            pltpu.VMEM((k,), jnp.int32),
            pltpu.SemaphoreType.DMA,
            pltpu.SemaphoreType.DMA,
            pltpu.SemaphoreType.DMA,
            pltpu.SemaphoreType.DMA,
            pltpu.SemaphoreType.DMA,
        ],
        compiler_params=pltpu.CompilerParams(needs_layout_passes=False),
        interpret=interpret,
    )

    return sck


_kern = _make_sc_kernel(128, 32768, 512)


def kernel(x):
    return _kern(x)
